# Initial kernel scaffold; baseline (speedup 1.0000x reference)
import jax
import jax.numpy as jnp
from jax.experimental import pallas as pl


def kernel(x, node_anchor_adj, W1, b1, W2, b2, W3, b3):
    raise NotImplementedError("write your pallas kernel here")


# speedup vs baseline: 1.1579x; 1.1579x over previous
import functools

import jax
import jax.numpy as jnp
from jax.experimental import pallas as pl
from jax.experimental.pallas import tpu as pltpu

EPS = 1e-12
BN = 2000


def _pass_in(adj_ref, x_ref, t_ref, col_ref, ones_ref):
    i = pl.program_id(0)

    @pl.when(i == 0)
    def _init():
        t_ref[...] = jnp.zeros_like(t_ref)
        col_ref[...] = jnp.zeros_like(col_ref)
        ones_ref[...] = jnp.ones_like(ones_ref)

    adj = adj_ref[...]
    t_ref[...] += jax.lax.dot_general(
        adj, x_ref[...], (((0,), (0,)), ((), ())),
        preferred_element_type=jnp.float32)
    col_ref[...] += jax.lax.dot_general(
        adj, ones_ref[...], (((0,), (0,)), ((), ())),
        preferred_element_type=jnp.float32)


def _pass_mid(t_ref, col_ref, w_ref, b_ref, adj_ref, tout_ref, v_ref):
    i = pl.program_id(0)

    @pl.when(i == 0)
    def _init():
        col = jnp.maximum(col_ref[:, 0:1], EPS)
        u = t_ref[...] / col
        v_ref[...] = jnp.dot(u, w_ref[...], preferred_element_type=jnp.float32)
        tout_ref[...] = jnp.zeros_like(tout_ref)

    adj = adj_ref[...]
    row = jnp.maximum(jnp.sum(adj, axis=1, keepdims=True), EPS)
    y = jnp.dot(adj, v_ref[...], preferred_element_type=jnp.float32)
    g = jnp.maximum(y / row + b_ref[...], 0.0)
    tout_ref[...] += jax.lax.dot_general(
        adj, g, (((0,), (0,)), ((), ())),
        preferred_element_type=jnp.float32)


def _pass_out(t_ref, col_ref, w_ref, b_ref, adj_ref, out_ref, v_ref):
    i = pl.program_id(0)

    @pl.when(i == 0)
    def _init():
        col = jnp.maximum(col_ref[:, 0:1], EPS)
        u = t_ref[...] / col
        v_ref[...] = jnp.dot(u, w_ref[...], preferred_element_type=jnp.float32)

    adj = adj_ref[...]
    row = jnp.maximum(jnp.sum(adj, axis=1, keepdims=True), EPS)
    y = jnp.dot(adj, v_ref[...], preferred_element_type=jnp.float32)
    out_ref[...] = y / row + b_ref[...]


@jax.jit
def kernel(x, node_anchor_adj, W1, b1, W2, b2, W3, b3):
    n, nfeat = x.shape
    a = node_anchor_adj.shape[1]
    nblk = n // BN
    adj = node_anchor_adj

    t0, col = pl.pallas_call(
        _pass_in,
        grid=(nblk,),
        in_specs=[
            pl.BlockSpec((BN, a), lambda i: (i, 0)),
            pl.BlockSpec((BN, nfeat), lambda i: (i, 0)),
        ],
        out_specs=[
            pl.BlockSpec((a, nfeat), lambda i: (0, 0)),
            pl.BlockSpec((a, 8), lambda i: (0, 0)),
        ],
        out_shape=[
            jax.ShapeDtypeStruct((a, nfeat), jnp.float32),
            jax.ShapeDtypeStruct((a, 8), jnp.float32),
        ],
        scratch_shapes=[pltpu.VMEM((BN, 8), jnp.float32)],
    )(adj, x)

    def mid(t, W, b):
        fin = W.shape[0]
        fout = W.shape[1]
        return pl.pallas_call(
            _pass_mid,
            grid=(nblk,),
            in_specs=[
                pl.BlockSpec((a, fin), lambda i: (0, 0)),
                pl.BlockSpec((a, 8), lambda i: (0, 0)),
                pl.BlockSpec((fin, fout), lambda i: (0, 0)),
                pl.BlockSpec((1, fout), lambda i: (0, 0)),
                pl.BlockSpec((BN, a), lambda i: (i, 0)),
            ],
            out_specs=pl.BlockSpec((a, fout), lambda i: (0, 0)),
            out_shape=jax.ShapeDtypeStruct((a, fout), jnp.float32),
            scratch_shapes=[pltpu.VMEM((a, fout), jnp.float32)],
        )(t, col, W, b.reshape(1, fout), adj)

    t1 = mid(t0, W1, b1)
    t2 = mid(t1, W2, b2)

    fin, fout = W3.shape
    out = pl.pallas_call(
        _pass_out,
        grid=(nblk,),
        in_specs=[
            pl.BlockSpec((a, fin), lambda i: (0, 0)),
            pl.BlockSpec((a, 8), lambda i: (0, 0)),
            pl.BlockSpec((fin, fout), lambda i: (0, 0)),
            pl.BlockSpec((1, fout), lambda i: (0, 0)),
            pl.BlockSpec((BN, a), lambda i: (i, 0)),
        ],
        out_specs=pl.BlockSpec((BN, fout), lambda i: (i, 0)),
        out_shape=jax.ShapeDtypeStruct((n, fout), jnp.float32),
        scratch_shapes=[pltpu.VMEM((a, fout), jnp.float32)],
    )(t2, col, W3, b3.reshape(1, fout), adj)
    return out


# speedup vs baseline: 1.1793x; 1.0185x over previous
import functools

import jax
import jax.numpy as jnp
from jax.experimental import pallas as pl
from jax.experimental.pallas import tpu as pltpu

EPS = 1e-12
BN = 2000


def _pass_in(adj_ref, x_ref, t_ref, col_ref, adjb_ref, ones_ref):
    i = pl.program_id(0)

    @pl.when(i == 0)
    def _init():
        t_ref[...] = jnp.zeros_like(t_ref)
        col_ref[...] = jnp.zeros_like(col_ref)
        ones_ref[...] = jnp.ones_like(ones_ref)

    adj = adj_ref[...]
    adjb_ref[...] = adj.astype(jnp.bfloat16)
    t_ref[...] += jax.lax.dot_general(
        adj, x_ref[...], (((0,), (0,)), ((), ())),
        preferred_element_type=jnp.float32)
    col_ref[...] += jax.lax.dot_general(
        adj, ones_ref[...], (((0,), (0,)), ((), ())),
        preferred_element_type=jnp.float32)


def _pass_mid(t_ref, col_ref, w_ref, b_ref, adj_ref, tout_ref, v_ref):
    i = pl.program_id(0)

    @pl.when(i == 0)
    def _init():
        col = jnp.maximum(col_ref[:, 0:1], EPS)
        u = t_ref[...] / col
        v_ref[...] = jnp.dot(
            u, w_ref[...], preferred_element_type=jnp.float32
        ).astype(jnp.bfloat16)
        tout_ref[...] = jnp.zeros_like(tout_ref)

    adj = adj_ref[...]
    row = jnp.maximum(
        jnp.sum(adj.astype(jnp.float32), axis=1, keepdims=True), EPS)
    y = jnp.dot(adj, v_ref[...], preferred_element_type=jnp.float32)
    g = jnp.maximum(y / row + b_ref[...], 0.0)
    tout_ref[...] += jax.lax.dot_general(
        adj, g.astype(jnp.bfloat16), (((0,), (0,)), ((), ())),
        preferred_element_type=jnp.float32)


def _pass_out(t_ref, col_ref, w_ref, b_ref, adj_ref, out_ref, v_ref):
    i = pl.program_id(0)

    @pl.when(i == 0)
    def _init():
        col = jnp.maximum(col_ref[:, 0:1], EPS)
        u = t_ref[...] / col
        v_ref[...] = jnp.dot(
            u, w_ref[...], preferred_element_type=jnp.float32
        ).astype(jnp.bfloat16)

    adj = adj_ref[...]
    row = jnp.maximum(
        jnp.sum(adj.astype(jnp.float32), axis=1, keepdims=True), EPS)
    y = jnp.dot(adj, v_ref[...], preferred_element_type=jnp.float32)
    out_ref[...] = y / row + b_ref[...]


@jax.jit
def kernel(x, node_anchor_adj, W1, b1, W2, b2, W3, b3):
    n, nfeat = x.shape
    a = node_anchor_adj.shape[1]
    nblk = n // BN
    adj = node_anchor_adj

    t0, col, adjb = pl.pallas_call(
        _pass_in,
        grid=(nblk,),
        in_specs=[
            pl.BlockSpec((BN, a), lambda i: (i, 0)),
            pl.BlockSpec((BN, nfeat), lambda i: (i, 0)),
        ],
        out_specs=[
            pl.BlockSpec((a, nfeat), lambda i: (0, 0)),
            pl.BlockSpec((a, 8), lambda i: (0, 0)),
            pl.BlockSpec((BN, a), lambda i: (i, 0)),
        ],
        out_shape=[
            jax.ShapeDtypeStruct((a, nfeat), jnp.float32),
            jax.ShapeDtypeStruct((a, 8), jnp.float32),
            jax.ShapeDtypeStruct((n, a), jnp.bfloat16),
        ],
        scratch_shapes=[pltpu.VMEM((BN, 8), jnp.float32)],
    )(adj, x)

    def mid(t, W, b):
        fin = W.shape[0]
        fout = W.shape[1]
        return pl.pallas_call(
            _pass_mid,
            grid=(nblk,),
            in_specs=[
                pl.BlockSpec((a, fin), lambda i: (0, 0)),
                pl.BlockSpec((a, 8), lambda i: (0, 0)),
                pl.BlockSpec((fin, fout), lambda i: (0, 0)),
                pl.BlockSpec((1, fout), lambda i: (0, 0)),
                pl.BlockSpec((BN, a), lambda i: (i, 0)),
            ],
            out_specs=pl.BlockSpec((a, fout), lambda i: (0, 0)),
            out_shape=jax.ShapeDtypeStruct((a, fout), jnp.float32),
            scratch_shapes=[pltpu.VMEM((a, fout), jnp.bfloat16)],
        )(t, col, W, b.reshape(1, fout), adjb)

    t1 = mid(t0, W1, b1)
    t2 = mid(t1, W2, b2)

    fin, fout = W3.shape
    out = pl.pallas_call(
        _pass_out,
        grid=(nblk,),
        in_specs=[
            pl.BlockSpec((a, fin), lambda i: (0, 0)),
            pl.BlockSpec((a, 8), lambda i: (0, 0)),
            pl.BlockSpec((fin, fout), lambda i: (0, 0)),
            pl.BlockSpec((1, fout), lambda i: (0, 0)),
            pl.BlockSpec((BN, a), lambda i: (i, 0)),
        ],
        out_specs=pl.BlockSpec((BN, fout), lambda i: (i, 0)),
        out_shape=jax.ShapeDtypeStruct((n, fout), jnp.float32),
        scratch_shapes=[pltpu.VMEM((a, fout), jnp.bfloat16)],
    )(t2, col, W3, b3.reshape(1, fout), adjb)
    return out


# speedup vs baseline: 1.2691x; 1.0761x over previous
import jax
import jax.numpy as jnp
from jax.experimental import pallas as pl
from jax.experimental.pallas import tpu as pltpu

EPS = 1e-12
BN = 2000

_DN0 = (((0,), (0,)), ((), ()))


def _pass_in(adj_ref, x_ref, t_ref, col_ref, adjb_ref, ones_ref):
    i = pl.program_id(0)

    @pl.when(i == 0)
    def _init():
        t_ref[...] = jnp.zeros_like(t_ref)
        col_ref[...] = jnp.zeros_like(col_ref)
        ones_ref[...] = jnp.ones_like(ones_ref)

    adj = adj_ref[...]
    adjb_ref[...] = adj.astype(jnp.bfloat16)
    t_ref[...] += jax.lax.dot_general(
        x_ref[...], adj, _DN0, preferred_element_type=jnp.float32)
    col_ref[...] += jax.lax.dot_general(
        ones_ref[...], adj, _DN0, preferred_element_type=jnp.float32)


def _pass_mid(t_ref, col_ref, w_ref, b_ref, adj_ref, tout_ref, v_ref, ones_ref):
    i = pl.program_id(0)

    @pl.when(i == 0)
    def _init():
        col = jnp.maximum(col_ref[0:1, :], EPS)
        u = t_ref[...] / col
        v_ref[...] = jax.lax.dot_general(
            u, w_ref[...], _DN0, preferred_element_type=jnp.float32
        ).astype(jnp.bfloat16)
        tout_ref[...] = jnp.zeros_like(tout_ref)
        ones_ref[...] = jnp.ones_like(ones_ref)

    adj = adj_ref[...]
    row = jnp.maximum(
        jnp.dot(adj, ones_ref[...],
                preferred_element_type=jnp.float32)[:, 0:1], EPS)
    y = jnp.dot(adj, v_ref[...], preferred_element_type=jnp.float32)
    g = jnp.maximum(y / row + b_ref[...], 0.0)
    tout_ref[...] += jax.lax.dot_general(
        g.astype(jnp.bfloat16), adj, _DN0, preferred_element_type=jnp.float32)


def _pass_out(t_ref, col_ref, w_ref, b_ref, adj_ref, out_ref, v_ref, ones_ref):
    i = pl.program_id(0)

    @pl.when(i == 0)
    def _init():
        col = jnp.maximum(col_ref[0:1, :], EPS)
        u = t_ref[...] / col
        v_ref[...] = jax.lax.dot_general(
            u, w_ref[...], _DN0, preferred_element_type=jnp.float32
        ).astype(jnp.bfloat16)
        ones_ref[...] = jnp.ones_like(ones_ref)

    adj = adj_ref[...]
    row = jnp.maximum(
        jnp.dot(adj, ones_ref[...],
                preferred_element_type=jnp.float32)[:, 0:1], EPS)
    y = jnp.dot(adj, v_ref[...], preferred_element_type=jnp.float32)
    out_ref[...] = y / row + b_ref[...]


@jax.jit
def kernel(x, node_anchor_adj, W1, b1, W2, b2, W3, b3):
    n, nfeat = x.shape
    a = node_anchor_adj.shape[1]
    nblk = n // BN
    adj = node_anchor_adj

    t0, col, adjb = pl.pallas_call(
        _pass_in,
        grid=(nblk,),
        in_specs=[
            pl.BlockSpec((BN, a), lambda i: (i, 0)),
            pl.BlockSpec((BN, nfeat), lambda i: (i, 0)),
        ],
        out_specs=[
            pl.BlockSpec((nfeat, a), lambda i: (0, 0)),
            pl.BlockSpec((8, a), lambda i: (0, 0)),
            pl.BlockSpec((BN, a), lambda i: (i, 0)),
        ],
        out_shape=[
            jax.ShapeDtypeStruct((nfeat, a), jnp.float32),
            jax.ShapeDtypeStruct((8, a), jnp.float32),
            jax.ShapeDtypeStruct((n, a), jnp.bfloat16),
        ],
        scratch_shapes=[pltpu.VMEM((BN, 8), jnp.float32)],
    )(adj, x)

    def mid(t, W, b):
        fin, fout = W.shape
        return pl.pallas_call(
            _pass_mid,
            grid=(nblk,),
            in_specs=[
                pl.BlockSpec((fin, a), lambda i: (0, 0)),
                pl.BlockSpec((8, a), lambda i: (0, 0)),
                pl.BlockSpec((fin, fout), lambda i: (0, 0)),
                pl.BlockSpec((1, fout), lambda i: (0, 0)),
                pl.BlockSpec((BN, a), lambda i: (i, 0)),
            ],
            out_specs=pl.BlockSpec((fout, a), lambda i: (0, 0)),
            out_shape=jax.ShapeDtypeStruct((fout, a), jnp.float32),
            scratch_shapes=[
                pltpu.VMEM((a, fout), jnp.bfloat16),
                pltpu.VMEM((a, 8), jnp.bfloat16),
            ],
        )(t, col, W, b.reshape(1, fout), adjb)

    t1 = mid(t0, W1, b1)
    t2 = mid(t1, W2, b2)

    fin, fout = W3.shape
    out = pl.pallas_call(
        _pass_out,
        grid=(nblk,),
        in_specs=[
            pl.BlockSpec((fin, a), lambda i: (0, 0)),
            pl.BlockSpec((8, a), lambda i: (0, 0)),
            pl.BlockSpec((fin, fout), lambda i: (0, 0)),
            pl.BlockSpec((1, fout), lambda i: (0, 0)),
            pl.BlockSpec((BN, a), lambda i: (i, 0)),
        ],
        out_specs=pl.BlockSpec((BN, fout), lambda i: (i, 0)),
        out_shape=jax.ShapeDtypeStruct((n, fout), jnp.float32),
        scratch_shapes=[
            pltpu.VMEM((a, fout), jnp.bfloat16),
            pltpu.VMEM((a, 8), jnp.bfloat16),
        ],
    )(t2, col, W3, b3.reshape(1, fout), adjb)
    return out


# speedup vs baseline: 1.3407x; 1.0564x over previous
import functools

import jax
import jax.numpy as jnp
from jax.experimental import pallas as pl
from jax.experimental.pallas import tpu as pltpu

EPS = 1e-12
BN = 2000
BN2 = 5000

_DN0 = (((0,), (0,)), ((), ()))


def _pass_in(adj_ref, x_ref, t_ref, col_ref, adjb_ref, ones_ref):
    i = pl.program_id(0)

    @pl.when(i == 0)
    def _init():
        t_ref[...] = jnp.zeros_like(t_ref)
        col_ref[...] = jnp.zeros_like(col_ref)
        ones_ref[...] = jnp.ones_like(ones_ref)

    adj = adj_ref[...]
    adjb_ref[...] = adj.astype(jnp.bfloat16)
    t_ref[...] += jax.lax.dot_general(
        x_ref[...], adj, _DN0, preferred_element_type=jnp.float32)
    col_ref[...] += jax.lax.dot_general(
        ones_ref[...], adj, _DN0, preferred_element_type=jnp.float32)


def _mkv(t, col_ref, w):
    col = jnp.maximum(col_ref[0:1, :], EPS)
    u = t / col
    return jax.lax.dot_general(
        u, w, _DN0, preferred_element_type=jnp.float32
    ).astype(jnp.bfloat16)


def _pass_fused(nblk, t0_ref, col_ref, ws_ref, bs_ref, adj_ref, out_ref,
                v_ref, ones_ref, t1_ref, t2_ref):
    i = pl.program_id(0)
    ph = i // nblk

    @pl.when(i == 0)
    def _init0():
        ones_ref[...] = jnp.ones_like(ones_ref)
        t1_ref[...] = jnp.zeros_like(t1_ref)
        t2_ref[...] = jnp.zeros_like(t2_ref)
        v_ref[...] = _mkv(t0_ref[...], col_ref, ws_ref[0])

    @pl.when(i == nblk)
    def _init1():
        v_ref[...] = _mkv(t1_ref[...], col_ref, ws_ref[1])

    @pl.when(i == 2 * nblk)
    def _init2():
        v_ref[...] = _mkv(t2_ref[...], col_ref, ws_ref[2])

    adj = adj_ref[...]
    row = jnp.maximum(
        jnp.dot(adj, ones_ref[...],
                preferred_element_type=jnp.float32)[:, 0:1], EPS)
    y = jnp.dot(adj, v_ref[...], preferred_element_type=jnp.float32)
    z = y / row + bs_ref[pl.ds(ph, 1), :]

    @pl.when(ph == 0)
    def _acc1():
        g = jnp.maximum(z, 0.0).astype(jnp.bfloat16)
        t1_ref[...] += jax.lax.dot_general(
            g, adj, _DN0, preferred_element_type=jnp.float32)

    @pl.when(ph == 1)
    def _acc2():
        g = jnp.maximum(z, 0.0).astype(jnp.bfloat16)
        t2_ref[...] += jax.lax.dot_general(
            g, adj, _DN0, preferred_element_type=jnp.float32)

    @pl.when(ph == 2)
    def _emit():
        out_ref[...] = z[:, :out_ref.shape[1]]


@jax.jit
def kernel(x, node_anchor_adj, W1, b1, W2, b2, W3, b3):
    n, nfeat = x.shape
    a = node_anchor_adj.shape[1]
    nblk = n // BN
    adj = node_anchor_adj

    t0, col, adjb = pl.pallas_call(
        _pass_in,
        grid=(nblk,),
        in_specs=[
            pl.BlockSpec((BN, a), lambda i: (i, 0)),
            pl.BlockSpec((BN, nfeat), lambda i: (i, 0)),
        ],
        out_specs=[
            pl.BlockSpec((nfeat, a), lambda i: (0, 0)),
            pl.BlockSpec((8, a), lambda i: (0, 0)),
            pl.BlockSpec((BN, a), lambda i: (i, 0)),
        ],
        out_shape=[
            jax.ShapeDtypeStruct((nfeat, a), jnp.float32),
            jax.ShapeDtypeStruct((8, a), jnp.float32),
            jax.ShapeDtypeStruct((n, a), jnp.bfloat16),
        ],
        scratch_shapes=[pltpu.VMEM((BN, 8), jnp.float32)],
    )(adj, x)

    nh = W1.shape[1]
    nclass = W3.shape[1]
    ws = jnp.stack([W1, W2,
                    jnp.pad(W3, ((0, 0), (0, nh - nclass)))])
    bs = jnp.stack([b1, b2, jnp.pad(b3, (0, nh - nclass))])

    nblk2 = n // BN2
    out = pl.pallas_call(
        functools.partial(_pass_fused, nblk2),
        grid=(3 * nblk2,),
        in_specs=[
            pl.BlockSpec((nh, a), lambda i: (0, 0)),
            pl.BlockSpec((8, a), lambda i: (0, 0)),
            pl.BlockSpec((3, nh, nh), lambda i: (0, 0, 0)),
            pl.BlockSpec((3, nh), lambda i: (0, 0)),
            pl.BlockSpec((BN2, a), lambda i: (i % (n // BN2), 0)),
        ],
        out_specs=pl.BlockSpec(
            (BN2, nclass),
            lambda i: (jnp.maximum(i - 2 * (n // BN2), 0), 0)),
        out_shape=jax.ShapeDtypeStruct((n, nclass), jnp.float32),
        scratch_shapes=[
            pltpu.VMEM((a, nh), jnp.bfloat16),
            pltpu.VMEM((a, 8), jnp.bfloat16),
            pltpu.VMEM((nh, a), jnp.float32),
            pltpu.VMEM((nh, a), jnp.float32),
        ],
    )(t0, col, ws, bs, adjb)
    return out
